# baseline (speedup 1.0000x reference)
import jax
import jax.numpy as jnp
from jax import lax
from jax.experimental import pallas as pl


_B, _S, _C = 1024, 50, 1000
_CB = 120


def _onehot_t_body(xt_ref, out_ref):
    c0 = pl.program_id(0) * _CB
    xt = xt_ref[...]
    cvals = c0 + lax.broadcasted_iota(jnp.int32, (_S, _CB, _B), 1)
    out_ref[...] = (xt[:, None, :] == cvals).astype(jnp.float32)


def kernel(inputs):
    xt = inputs.astype(jnp.int32).T
    out_t = pl.pallas_call(
        _onehot_t_body,
        grid=(-(-_C // _CB),),
        in_specs=[pl.BlockSpec((_S, _B), lambda i: (0, 0))],
        out_specs=pl.BlockSpec((_S, _CB, _B), lambda i: (0, i, 0)),
        out_shape=jax.ShapeDtypeStruct((_S, _C, _B), jnp.float32),
    )(xt)
    return out_t.transpose(2, 0, 1)


# speedup vs baseline: 1.0565x; 1.0565x over previous
import jax
import jax.numpy as jnp
from jax import lax
from jax.experimental import pallas as pl


_B, _S, _C = 1024, 50, 1000
_CB = 48


def _onehot_t_body(xt_ref, out_ref):
    c0 = pl.program_id(0) * _CB
    xt = xt_ref[...]
    cvals = c0 + lax.broadcasted_iota(jnp.int32, (_S, _CB, _B), 1)
    out_ref[...] = (xt[:, None, :] == cvals).astype(jnp.float32)


def kernel(inputs):
    xt = inputs.astype(jnp.int32).T
    out_t = pl.pallas_call(
        _onehot_t_body,
        grid=(-(-_C // _CB),),
        in_specs=[pl.BlockSpec((_S, _B), lambda i: (0, 0))],
        out_specs=pl.BlockSpec((_S, _CB, _B), lambda i: (0, i, 0)),
        out_shape=jax.ShapeDtypeStruct((_S, _C, _B), jnp.float32),
    )(xt)
    return out_t.transpose(2, 0, 1)


# speedup vs baseline: 1.0814x; 1.0236x over previous
import jax
import jax.numpy as jnp
from jax import lax
from jax.experimental import pallas as pl


_B, _S, _C = 1024, 50, 1000
_CB = 40


def _onehot_t_body(xt_ref, out_ref):
    c0 = pl.program_id(0) * _CB
    xt = xt_ref[...]
    cvals = c0 + lax.broadcasted_iota(jnp.int32, (_S, _CB, _B), 1)
    out_ref[...] = (xt[:, None, :] == cvals).astype(jnp.float32)


def kernel(inputs):
    xt = inputs.astype(jnp.int32).T
    out_t = pl.pallas_call(
        _onehot_t_body,
        grid=(-(-_C // _CB),),
        in_specs=[pl.BlockSpec((_S, _B), lambda i: (0, 0))],
        out_specs=pl.BlockSpec((_S, _CB, _B), lambda i: (0, i, 0)),
        out_shape=jax.ShapeDtypeStruct((_S, _C, _B), jnp.float32),
    )(xt)
    return out_t.transpose(2, 0, 1)
